# baseline (speedup 1.0000x reference)
import functools

import jax
import jax.numpy as jnp
import numpy as np
from jax.experimental import pallas as pl
from jax.experimental.pallas import tpu as pltpu

_N_BINS = 20
_NB_PAD = 32
_ROWS = 8000
_DEPTH = 4


def _make_bounds_col():
    b = np.full((_NB_PAD, 128), np.inf, dtype=np.float32)
    b[: _N_BINS + 1, 0] = np.linspace(0.0, 1.0, _N_BINS + 1).astype(np.float32)
    return b


def _ece_body(x_hbm, lab_hbm, b_ref, out_ref, xbuf, lbuf, xsem, lsem):
    n, c = x_hbm.shape
    nchunk = n // _ROWS

    def xcopy(i, slot):
        return pltpu.make_async_copy(
            x_hbm.at[pl.ds(i * _ROWS, _ROWS), :], xbuf.at[slot], xsem.at[slot])

    def lcopy(i, slot):
        return pltpu.make_async_copy(lab_hbm.at[i], lbuf.at[slot], lsem.at[slot])

    for d in range(min(_DEPTH, nchunk)):
        xcopy(d, d).start()
        lcopy(d, d).start()

    bounds = b_ref[...][:, 0:1]

    def loop(i, part):
        slot = jax.lax.rem(i, _DEPTH)
        xcopy(i, slot).wait()
        lcopy(i, slot).wait()
        x = xbuf[slot]
        lab = lbuf[slot]
        xt = x.T
        conf = jnp.max(xt, axis=0, keepdims=True)
        pred = jnp.argmax(xt, axis=0).astype(jnp.int32)[None, :]
        acc = (pred == lab).astype(jnp.float32)

        m = (conf > bounds).astype(jnp.float32)
        vt = jnp.concatenate([jnp.ones_like(conf), conf, acc], axis=0)
        upd = jax.lax.dot_general(
            vt, m, (((1,), (1,)), ((), ())), preferred_element_type=jnp.float32)

        @pl.when(i + _DEPTH < nchunk)
        def _():
            xcopy(i + _DEPTH, slot).start()
            lcopy(i + _DEPTH, slot).start()

        return part + upd

    part = jax.lax.fori_loop(
        0, nchunk, loop, jnp.zeros((3, _NB_PAD), jnp.float32))

    tot = jnp.pad(part, ((0, 0), (0, 128 - _NB_PAD)))
    cum_cnt = tot[0, :]
    cum_conf = tot[1, :]
    cum_acc = tot[2, :]

    def shift(v):
        return jnp.concatenate([v[1:], v[-1:]])

    col = jax.lax.broadcasted_iota(jnp.int32, (128,), 0)
    cnt = cum_cnt - shift(cum_cnt)
    sum_conf = cum_conf - shift(cum_conf)
    sum_acc = cum_acc - shift(cum_acc)
    in_range = col < _N_BINS
    cnt = jnp.where(in_range, cnt, 0.0)
    sum_conf = jnp.where(in_range, sum_conf, 0.0)
    sum_acc = jnp.where(in_range, sum_acc, 0.0)
    denom = jnp.maximum(cnt, 1.0)
    nonzero = cnt > 0.0
    acc_bin = jnp.where(nonzero, sum_acc / denom, 0.0)
    conf_bin = jnp.where(nonzero, sum_conf / denom, 0.0)
    prop = cnt / float(n)
    ece = jnp.sum(jnp.where(nonzero, jnp.abs(conf_bin - acc_bin) * prop, 0.0))
    out_ref[...] = jnp.concatenate(
        [jnp.full((1, 128), ece, jnp.float32), acc_bin[None, :],
         jnp.zeros((6, 128), jnp.float32)], axis=0)


@jax.jit
def _ece_pallas(softmaxes, labels):
    n, c = softmaxes.shape
    nchunk = n // _ROWS
    lab3 = labels.reshape(nchunk, 1, _ROWS)
    bounds_col = jnp.asarray(_make_bounds_col())
    out = pl.pallas_call(
        _ece_body,
        in_specs=[
            pl.BlockSpec(memory_space=pltpu.HBM),
            pl.BlockSpec(memory_space=pltpu.HBM),
            pl.BlockSpec(memory_space=pltpu.VMEM),
        ],
        out_specs=pl.BlockSpec(memory_space=pltpu.VMEM),
        out_shape=jax.ShapeDtypeStruct((8, 128), jnp.float32),
        scratch_shapes=[
            pltpu.VMEM((_DEPTH, _ROWS, 100), jnp.float32),
            pltpu.VMEM((_DEPTH, 1, _ROWS), jnp.int32),
            pltpu.SemaphoreType.DMA((_DEPTH,)),
            pltpu.SemaphoreType.DMA((_DEPTH,)),
        ],
    )(softmaxes, lab3, bounds_col)
    ece = out[0, 0:1]
    ys = out[1, :_N_BINS]
    return ece, ys


def kernel(softmaxes, labels):
    return _ece_pallas(softmaxes, labels)
